# Initial kernel scaffold; baseline (speedup 1.0000x reference)
import jax
import jax.numpy as jnp
from jax.experimental import pallas as pl


def kernel(fc_x, sc_x, fc_edge_attr, sc_edge_attr, edge_index, params):
    raise NotImplementedError("write your pallas kernel here")


# speedup vs baseline: 5.4847x; 5.4847x over previous
import functools
import math

import jax
import jax.numpy as jnp
from jax import lax
from jax.experimental import pallas as pl
from jax.experimental.pallas import tpu as pltpu
from jax.experimental.pallas import tpu_sc as plsc

_NW = 32
_CH = 80
_BN = 1.0 / math.sqrt(1.0 + 1e-5)
_SC_PARAMS = pltpu.CompilerParams(use_tc_tiling_on_sc=False)


def _relu(x):
    return jnp.maximum(x, 0.0)


def _node0_body(fc_x, sc_x, fnw, snw, fewf, fews, g_ref, h0_ref):
    h0f = _relu(fc_x[...] @ fnw[...])
    h0s = _relu(sc_x[...] @ snw[...])
    gf = h0f @ fewf[...]
    gs = h0s @ fews[...]
    g_ref[...] = jnp.concatenate([gf, gs], axis=1)
    h0_ref[...] = jnp.concatenate([h0f, h0s], axis=1)


def _node0(fc_x, sc_x, p):
    N = fc_x.shape[0]
    return pl.pallas_call(
        _node0_body,
        out_shape=(
            jax.ShapeDtypeStruct((N, 32), jnp.float32),
            jax.ShapeDtypeStruct((N, 16), jnp.float32),
        ),
    )(fc_x, sc_x, p["fc0_node_w"], p["sc0_node_w"],
      p["fc0_fe_w"][:8], p["sc0_fe_w"][:8])


def _sc_gather_sum(g, src, dst):
    E = src.shape[0]
    per_w = E // _NW
    n_ch = per_w // _CH
    mesh = plsc.VectorSubcoreMesh(core_axis_name="c", subcore_axis_name="s")

    @functools.partial(
        pl.kernel, mesh=mesh, compiler_params=_SC_PARAMS,
        out_type=jax.ShapeDtypeStruct((E, 32), jnp.float32),
        scratch_types=[
            pltpu.VMEM((_CH,), jnp.int32),
            pltpu.VMEM((_CH,), jnp.int32),
            pltpu.VMEM((_CH, 32), jnp.float32),
            pltpu.VMEM((_CH, 32), jnp.float32),
            pltpu.SemaphoreType.DMA,
            pltpu.SemaphoreType.DMA,
        ],
    )
    def k(g_hbm, src_hbm, dst_hbm, out_hbm, si, di, ra, rb, s1, s2):
        wid = lax.axis_index("s") * 2 + lax.axis_index("c")
        base = wid * per_w

        @pl.loop(0, n_ch)
        def _(ci):
            b = base + ci * _CH
            pltpu.sync_copy(src_hbm.at[pl.ds(b, _CH)], si)
            pltpu.sync_copy(dst_hbm.at[pl.ds(b, _CH)], di)
            ca = pltpu.async_copy(g_hbm.at[si], ra, s1)
            cb = pltpu.async_copy(g_hbm.at[di], rb, s2)
            ca.wait()
            cb.wait()

            @pl.loop(0, _CH)
            def _(r):
                for c in (0, 16):
                    slc = (pl.ds(r, 1), pl.ds(c, 16))
                    ra.at[*slc][...] = ra.at[*slc][...] + rb.at[*slc][...]

            pltpu.sync_copy(ra, out_hbm.at[pl.ds(b, _CH)])

    return k(g, src, dst)


def _edge1_body(gs_ref, fcea_ref, scea_ref, few, sew, fewf, febf, fews, febs,
                ewf1, ews1, out_ref):
    gs = gs_ref[...]
    fcea = fcea_ref[...]
    scea = scea_ref[...]
    p = _relu(fcea)
    q = _relu(-fcea)
    wp = jnp.maximum(few[...], 0.0)
    wm = jnp.maximum(-few[...], 0.0)
    e1f = p * wp + q * wm
    sw = sew[...]
    e1s = _relu(scea[:, 0:1] * sw[0:1] + scea[:, 1:2] * sw[1:2]
                + scea[:, 2:3] * sw[2:3])
    cf = e1f @ fewf[...] + febf[...]
    e2f = _relu(gs[:, :16] + cf)
    e11f = _relu(e2f @ ewf1[...])
    cs = e1s @ fews[...] + febs[...]
    e2s = _relu(gs[:, 16:] + cs)
    e11s = _relu(e2s @ ews1[...])
    z2 = jnp.zeros_like(e11f)
    out_ref[...] = jnp.concatenate([e1s, p, q, e11f, e11s, z2], axis=1)


def _edge1(gsum, fc_ea, sc_ea, p, blk):
    E = gsum.shape[0]
    grid = E // blk
    full = lambda i: (0, 0)
    return pl.pallas_call(
        _edge1_body,
        grid=(grid,),
        in_specs=[
            pl.BlockSpec((blk, 32), lambda i: (i, 0)),
            pl.BlockSpec((blk, 1), lambda i: (i, 0)),
            pl.BlockSpec((blk, 3), lambda i: (i, 0)),
            pl.BlockSpec((1, 8), full),
            pl.BlockSpec((3, 8), full),
            pl.BlockSpec((8, 16), full),
            pl.BlockSpec((1, 16), full),
            pl.BlockSpec((8, 16), full),
            pl.BlockSpec((1, 16), full),
            pl.BlockSpec((16, 2), full),
            pl.BlockSpec((16, 2), full),
        ],
        out_specs=pl.BlockSpec((blk, 16), lambda i: (i, 0)),
        out_shape=jax.ShapeDtypeStruct((E, 16), jnp.float32),
    )(gsum, fc_ea, sc_ea,
      p["fc0_edge_w"], p["sc0_edge_w"],
      p["fc0_fe_w"][8:], p["fc0_fe_b"].reshape(1, 16),
      p["sc0_fe_w"][8:], p["sc0_fe_b"].reshape(1, 16),
      p["fc1_edge_w"], p["sc1_edge_w"])


def _sc_scatter_add(scat, dst, zacc):
    E, W = scat.shape
    N = zacc.shape[0]
    e_core = E // 2
    per_w = e_core // 16
    n_ch = per_w // _CH
    rows_w = N // 16
    mesh = plsc.VectorSubcoreMesh(core_axis_name="c", subcore_axis_name="s")

    @functools.partial(
        pl.kernel, mesh=mesh, compiler_params=_SC_PARAMS,
        out_type=jax.ShapeDtypeStruct((2, N, W), jnp.float32),
        scratch_types=[
            pltpu.VMEM((_CH,), jnp.int32),
            pltpu.VMEM((_CH, W), jnp.float32),
            pltpu.VMEM_SHARED((N, W), jnp.float32),
        ],
    )
    def k(scat_hbm, dst_hbm, z_hbm, out_hbm, idx, rows, acc):
        c = lax.axis_index("c")
        s = lax.axis_index("s")
        r0 = s * rows_w
        pltpu.sync_copy(z_hbm.at[pl.ds(r0, rows_w)], acc.at[pl.ds(r0, rows_w)])
        plsc.subcore_barrier()
        base = c * e_core + s * per_w

        @pl.loop(0, n_ch)
        def _(ci):
            b = base + ci * _CH
            pltpu.sync_copy(dst_hbm.at[pl.ds(b, _CH)], idx)
            pltpu.sync_copy(scat_hbm.at[pl.ds(b, _CH)], rows)
            pltpu.sync_copy(rows, acc.at[idx], add=True)

        plsc.subcore_barrier()
        pltpu.sync_copy(acc.at[pl.ds(r0, rows_w)],
                        out_hbm.at[c, pl.ds(r0, rows_w)])

    return k(scat, dst, zacc)


def _node1_body(parts_ref, h0_ref, few,
                ffn0w, ffn0b, fbn0g, fbn0b, fnw1, ffn1w, ffn1b, fbn1g, fbn1b,
                sfn0w, sfn0b, sbn0g, sbn0b, snw1, sfn1w, sfn1b, sbn1g, sbn1b,
                out_ref):
    parts = parts_ref[...]
    a = parts[0] + parts[1]
    h0 = h0_ref[...]
    wp = jnp.maximum(few[...], 0.0)
    wm = jnp.maximum(-few[...], 0.0)
    fc_aggr0 = a[:, 8:9] * wp + a[:, 9:10] * wm
    sc_aggr0 = a[:, 0:8]
    fc_a1 = a[:, 10:12]
    sc_a1 = a[:, 12:14]

    def half(aggr0, h0e, a1, fn0w, fn0b, bn0g, bn0b, nw1, fn1w, fn1b,
             bn1g, bn1b):
        n2 = _relu(jnp.concatenate([aggr0, h0e], axis=1) @ fn0w[...]
                   + fn0b[...])
        x1 = n2 * (bn0g[...] * _BN) + bn0b[...]
        h1 = _relu(x1 @ nw1[...])
        n2b = _relu(jnp.concatenate([a1, h1], axis=1) @ fn1w[...]
                    + fn1b[...])
        return n2b * (bn1g[...] * _BN) + bn1b[...]

    zfc = half(fc_aggr0, h0[:, :8], fc_a1, ffn0w, ffn0b, fbn0g, fbn0b,
               fnw1, ffn1w, ffn1b, fbn1g, fbn1b)
    zsc = half(sc_aggr0, h0[:, 8:], sc_a1, sfn0w, sfn0b, sbn0g, sbn0b,
               snw1, sfn1w, sfn1b, sbn1g, sbn1b)
    pad = jnp.zeros_like(zfc)
    out_ref[...] = jnp.concatenate([zfc, zsc, pad, pad], axis=1)


def _node1(parts, h0, p):
    N = h0.shape[0]
    r2 = lambda name: p[name].reshape(1, -1)
    return pl.pallas_call(
        _node1_body,
        out_shape=jax.ShapeDtypeStruct((N, 16), jnp.float32),
    )(parts, h0, p["fc0_edge_w"],
      p["fc0_fn_w"], r2("fc0_fn_b"), r2("fc_bn0_g"), r2("fc_bn0_b"),
      p["fc1_node_w"], p["fc1_fn_w"], r2("fc1_fn_b"), r2("fc_bn1_g"),
      r2("fc_bn1_b"),
      p["sc0_fn_w"], r2("sc0_fn_b"), r2("sc_bn0_g"), r2("sc_bn0_b"),
      p["sc1_node_w"], p["sc1_fn_w"], r2("sc1_fn_b"), r2("sc_bn1_g"),
      r2("sc_bn1_b"))


def _sc_gather_mul(zp, src, dst):
    E = src.shape[0]
    per_w = E // _NW
    n_ch = per_w // _CH
    mesh = plsc.VectorSubcoreMesh(core_axis_name="c", subcore_axis_name="s")

    @functools.partial(
        pl.kernel, mesh=mesh, compiler_params=_SC_PARAMS,
        out_type=jax.ShapeDtypeStruct((E, 16), jnp.float32),
        scratch_types=[
            pltpu.VMEM((_CH,), jnp.int32),
            pltpu.VMEM((_CH,), jnp.int32),
            pltpu.VMEM((_CH, 16), jnp.float32),
            pltpu.VMEM((_CH, 16), jnp.float32),
            pltpu.SemaphoreType.DMA,
            pltpu.SemaphoreType.DMA,
        ],
    )
    def k(z_hbm, src_hbm, dst_hbm, out_hbm, si, di, ra, rb, s1, s2):
        wid = lax.axis_index("s") * 2 + lax.axis_index("c")
        base = wid * per_w

        @pl.loop(0, n_ch)
        def _(ci):
            b = base + ci * _CH
            pltpu.sync_copy(src_hbm.at[pl.ds(b, _CH)], si)
            pltpu.sync_copy(dst_hbm.at[pl.ds(b, _CH)], di)
            ca = pltpu.async_copy(z_hbm.at[si], ra, s1)
            cb = pltpu.async_copy(z_hbm.at[di], rb, s2)
            ca.wait()
            cb.wait()

            @pl.loop(0, _CH)
            def _(r):
                slc = (pl.ds(r, 1), pl.ds(0, 16))
                ra.at[*slc][...] = ra.at[*slc][...] * rb.at[*slc][...]

            pltpu.sync_copy(ra, out_hbm.at[pl.ds(b, _CH)])

    return k(zp, src, dst)


def _dec_body(h_ref, w1, b1, w2, b2, w3, b3, w4, b4, out_ref):
    x = h_ref[...][:, :8]
    x = _relu(x @ w1[...] + b1[...])
    x = _relu(x @ w2[...] + b2[...])
    x = _relu(x @ w3[...] + b3[...])
    x = x @ w4[...] + b4[...]
    out_ref[...] = jax.nn.sigmoid(x)


def _decode(h16, p, blk):
    E = h16.shape[0]
    grid = E // blk
    full = lambda i: (0, 0)
    return pl.pallas_call(
        _dec_body,
        grid=(grid,),
        in_specs=[
            pl.BlockSpec((blk, 16), lambda i: (i, 0)),
            pl.BlockSpec((8, 64), full),
            pl.BlockSpec((1, 64), full),
            pl.BlockSpec((64, 128), full),
            pl.BlockSpec((1, 128), full),
            pl.BlockSpec((128, 32), full),
            pl.BlockSpec((1, 32), full),
            pl.BlockSpec((32, 1), full),
            pl.BlockSpec((1, 1), full),
        ],
        out_specs=pl.BlockSpec((blk, 1), lambda i: (i, 0)),
        out_shape=jax.ShapeDtypeStruct((E, 1), jnp.float32),
    )(h16, p["dec_w1"], p["dec_b1"].reshape(1, -1),
      p["dec_w2"], p["dec_b2"].reshape(1, -1),
      p["dec_w3"], p["dec_b3"].reshape(1, -1),
      p["dec_w4"], p["dec_b4"].reshape(1, -1))


def kernel(fc_x, sc_x, fc_edge_attr, sc_edge_attr, edge_index, params):
    N = fc_x.shape[0]
    E = edge_index.shape[1]
    assert E % (_NW * _CH) == 0 and N % 16 == 0
    src = edge_index[0]
    dst = edge_index[1]

    g, h0 = _node0(fc_x, sc_x, params)
    gsum = _sc_gather_sum(g, src, dst)
    scat = _edge1(gsum, fc_edge_attr, sc_edge_attr, params, blk=6400)
    zacc = jnp.zeros((N, 16), jnp.float32)
    parts = _sc_scatter_add(scat, dst, zacc)
    zp = _node1(parts, h0, params)
    h16 = _sc_gather_mul(zp, src, dst)
    return _decode(h16, params, blk=6400)


# speedup vs baseline: 7.7427x; 1.4117x over previous
import functools
import math

import jax
import jax.numpy as jnp
from jax import lax
from jax.experimental import pallas as pl
from jax.experimental.pallas import tpu as pltpu
from jax.experimental.pallas import tpu_sc as plsc

_NW = 32
_CH = 80
_BN = 1.0 / math.sqrt(1.0 + 1e-5)
_SC_PARAMS = pltpu.CompilerParams(use_tc_tiling_on_sc=False)


def _relu(x):
    return jnp.maximum(x, 0.0)


def _node0_body(fc_x, sc_x, fnw, snw, fewf, fews, g_ref, h0_ref):
    h0f = _relu(fc_x[...] @ fnw[...])
    h0s = _relu(sc_x[...] @ snw[...])
    gf = h0f @ fewf[...]
    gs = h0s @ fews[...]
    g_ref[...] = jnp.concatenate([gf, gs], axis=1)
    h0_ref[...] = jnp.concatenate([h0f, h0s], axis=1)


def _node0(fc_x, sc_x, p):
    N = fc_x.shape[0]
    return pl.pallas_call(
        _node0_body,
        out_shape=(
            jax.ShapeDtypeStruct((N, 32), jnp.float32),
            jax.ShapeDtypeStruct((N, 16), jnp.float32),
        ),
    )(fc_x, sc_x, p["fc0_node_w"], p["sc0_node_w"],
      p["fc0_fe_w"][:8], p["sc0_fe_w"][:8])


def _sc_gather2(tbl, src_r, dst_r):
    W = tbl.shape[1]
    n_ch_tot, ch = src_r.shape
    E = n_ch_tot * ch
    per_w = n_ch_tot // _NW
    D = 5
    mesh = plsc.VectorSubcoreMesh(core_axis_name="c", subcore_axis_name="s")
    out_t = jax.ShapeDtypeStruct((E, W), jnp.float32)

    @functools.partial(
        pl.kernel, mesh=mesh, compiler_params=_SC_PARAMS,
        out_type=(out_t, out_t),
        scratch_types=(
            [pltpu.VMEM((per_w, ch), jnp.int32)] * 2
            + [pltpu.VMEM((ch, W), jnp.float32)] * (2 * D)
            + [pltpu.SemaphoreType.DMA] * (3 * D)
        ),
    )
    def k(tbl_hbm, src_hbm, dst_hbm, oa_hbm, ob_hbm, *scr):
        si_all, di_all = scr[0], scr[1]
        bufs = scr[2:2 + 2 * D]
        gsems = scr[2 + 2 * D:2 + 4 * D]
        wsems = scr[2 + 4 * D:2 + 5 * D]
        wid = lax.axis_index("s") * 2 + lax.axis_index("c")
        c0 = wid * per_w
        pltpu.sync_copy(src_hbm.at[pl.ds(c0, per_w)], si_all)
        pltpu.sync_copy(dst_hbm.at[pl.ds(c0, per_w)], di_all)

        def gathers(ci, d):
            pltpu.make_async_copy(
                tbl_hbm.at[si_all.at[ci]], bufs[2 * d], gsems[2 * d]).start()
            pltpu.make_async_copy(
                tbl_hbm.at[di_all.at[ci]], bufs[2 * d + 1],
                gsems[2 * d + 1]).start()

        def wait_gathers(ci, d):
            pltpu.make_async_copy(
                tbl_hbm.at[si_all.at[ci]], bufs[2 * d], gsems[2 * d]).wait()
            pltpu.make_async_copy(
                tbl_hbm.at[di_all.at[ci]], bufs[2 * d + 1],
                gsems[2 * d + 1]).wait()

        def writes(ci, d):
            b = (c0 + ci) * ch
            pltpu.make_async_copy(
                bufs[2 * d], oa_hbm.at[pl.ds(b, ch)], wsems[d]).start()
            pltpu.make_async_copy(
                bufs[2 * d + 1], ob_hbm.at[pl.ds(b, ch)], wsems[d]).start()

        def wait_writes(ci, d):
            b = (c0 + ci) * ch
            pltpu.make_async_copy(
                bufs[2 * d], oa_hbm.at[pl.ds(b, ch)], wsems[d]).wait()
            pltpu.make_async_copy(
                bufs[2 * d + 1], ob_hbm.at[pl.ds(b, ch)], wsems[d]).wait()

        for d in range(D):
            gathers(d, d)

        @pl.loop(0, per_w, step=D)
        def _(j0):
            for d in range(D):
                j = j0 + d
                wait_gathers(j, d)
                writes(j, d)

                @pl.when(j + D < per_w)
                def _():
                    wait_writes(j, d)
                    gathers(j + D, d)

        for d in range(D):
            wait_writes(per_w - D + d, d)

    return k(tbl, src_r, dst_r)


def _edge1_body(gs_ref, gd_ref, attr_ref, wa, wc, bc, w11, s1, out_ref):
    t1 = _relu(attr_ref[...] @ wa[...])
    cc = t1 @ wc[...] + bc[...]
    e2 = _relu(gs_ref[...] + gd_ref[...] + cc)
    out_ref[...] = _relu(e2 @ w11[...]) + t1 @ s1[...]


def _edge1(gsrc, gdst, attr_p, wa, wc, bc, w11, s1, blk):
    E4 = gsrc.shape[0]
    b4 = blk // 4
    grid = E4 // b4
    full = lambda i: (0, 0)
    return pl.pallas_call(
        _edge1_body,
        grid=(grid,),
        in_specs=[
            pl.BlockSpec((b4, 128), lambda i: (i, 0)),
            pl.BlockSpec((b4, 128), lambda i: (i, 0)),
            pl.BlockSpec((b4, 16), lambda i: (i, 0)),
            pl.BlockSpec((16, 128), full),
            pl.BlockSpec((128, 128), full),
            pl.BlockSpec((1, 128), full),
            pl.BlockSpec((128, 64), full),
            pl.BlockSpec((128, 64), full),
        ],
        out_specs=pl.BlockSpec((b4, 64), lambda i: (i, 0)),
        out_shape=jax.ShapeDtypeStruct((E4, 64), jnp.float32),
    )(gsrc, gdst, attr_p, wa, wc, bc, w11, s1)


def _edge1_weights(p):
    import jax.scipy.linalg as jsl
    wa = jnp.zeros((4, 32), jnp.float32)
    wa = wa.at[1:4, 0:8].set(p["sc0_edge_w"])
    wa = wa.at[0, 8].set(1.0)
    wa = wa.at[0, 9].set(-1.0)
    wa = wa.at[0, 10:18].set(p["fc0_edge_w"][0])
    wc = jnp.zeros((32, 32), jnp.float32)
    wc = wc.at[10:18, 0:16].set(p["fc0_fe_w"][8:])
    wc = wc.at[0:8, 16:32].set(p["sc0_fe_w"][8:])
    bc = jnp.concatenate([p["fc0_fe_b"], p["sc0_fe_b"]])
    w11 = jnp.zeros((32, 16), jnp.float32)
    w11 = w11.at[0:16, 10:12].set(p["fc1_edge_w"])
    w11 = w11.at[16:32, 12:14].set(p["sc1_edge_w"])
    s1 = jnp.zeros((32, 16), jnp.float32)
    s1 = s1.at[jnp.arange(10), jnp.arange(10)].set(1.0)
    bd = lambda m: jsl.block_diag(m, m, m, m)
    return (bd(wa), bd(wc), jnp.tile(bc, 4).reshape(1, 128),
            bd(w11), bd(s1))


def _sc_scatter_add(scat, dst, zacc):
    E, W = scat.shape
    N = zacc.shape[0]
    e_core = E // 2
    per_w = e_core // 16
    n_ch = per_w // _CH
    rows_w = N // 16
    mesh = plsc.VectorSubcoreMesh(core_axis_name="c", subcore_axis_name="s")

    @functools.partial(
        pl.kernel, mesh=mesh, compiler_params=_SC_PARAMS,
        out_type=jax.ShapeDtypeStruct((2, N, W), jnp.float32),
        scratch_types=[
            pltpu.VMEM((_CH,), jnp.int32),
            pltpu.VMEM((_CH, W), jnp.float32),
            pltpu.VMEM_SHARED((N, W), jnp.float32),
        ],
    )
    def k(scat_hbm, dst_hbm, z_hbm, out_hbm, idx, rows, acc):
        c = lax.axis_index("c")
        s = lax.axis_index("s")
        r0 = s * rows_w
        pltpu.sync_copy(z_hbm.at[pl.ds(r0, rows_w)], acc.at[pl.ds(r0, rows_w)])
        plsc.subcore_barrier()
        base = c * e_core + s * per_w

        @pl.loop(0, n_ch)
        def _(ci):
            b = base + ci * _CH
            pltpu.sync_copy(dst_hbm.at[pl.ds(b, _CH)], idx)
            pltpu.sync_copy(scat_hbm.at[pl.ds(b, _CH)], rows)
            pltpu.sync_copy(rows, acc.at[idx], add=True)

        plsc.subcore_barrier()
        pltpu.sync_copy(acc.at[pl.ds(r0, rows_w)],
                        out_hbm.at[c, pl.ds(r0, rows_w)])

    return k(scat, dst, zacc)


def _node1_body(parts_ref, h0_ref, few,
                ffn0w, ffn0b, fbn0g, fbn0b, fnw1, ffn1w, ffn1b, fbn1g, fbn1b,
                sfn0w, sfn0b, sbn0g, sbn0b, snw1, sfn1w, sfn1b, sbn1g, sbn1b,
                out_ref):
    parts = parts_ref[...]
    a = parts[0] + parts[1]
    h0 = h0_ref[...]
    wp = jnp.maximum(few[...], 0.0)
    wm = jnp.maximum(-few[...], 0.0)
    fc_aggr0 = a[:, 8:9] * wp + a[:, 9:10] * wm
    sc_aggr0 = a[:, 0:8]
    fc_a1 = a[:, 10:12]
    sc_a1 = a[:, 12:14]

    def half(aggr0, h0e, a1, fn0w, fn0b, bn0g, bn0b, nw1, fn1w, fn1b,
             bn1g, bn1b):
        n2 = _relu(jnp.concatenate([aggr0, h0e], axis=1) @ fn0w[...]
                   + fn0b[...])
        x1 = n2 * (bn0g[...] * _BN) + bn0b[...]
        h1 = _relu(x1 @ nw1[...])
        n2b = _relu(jnp.concatenate([a1, h1], axis=1) @ fn1w[...]
                    + fn1b[...])
        return n2b * (bn1g[...] * _BN) + bn1b[...]

    zfc = half(fc_aggr0, h0[:, :8], fc_a1, ffn0w, ffn0b, fbn0g, fbn0b,
               fnw1, ffn1w, ffn1b, fbn1g, fbn1b)
    zsc = half(sc_aggr0, h0[:, 8:], sc_a1, sfn0w, sfn0b, sbn0g, sbn0b,
               snw1, sfn1w, sfn1b, sbn1g, sbn1b)
    pad = jnp.zeros_like(zfc)
    out_ref[...] = jnp.concatenate([zfc, zsc, pad, pad], axis=1)


def _node1(parts, h0, p):
    N = h0.shape[0]
    r2 = lambda name: p[name].reshape(1, -1)
    return pl.pallas_call(
        _node1_body,
        out_shape=jax.ShapeDtypeStruct((N, 16), jnp.float32),
    )(parts, h0, p["fc0_edge_w"],
      p["fc0_fn_w"], r2("fc0_fn_b"), r2("fc_bn0_g"), r2("fc_bn0_b"),
      p["fc1_node_w"], p["fc1_fn_w"], r2("fc1_fn_b"), r2("fc_bn1_g"),
      r2("fc_bn1_b"),
      p["sc0_fn_w"], r2("sc0_fn_b"), r2("sc_bn0_g"), r2("sc_bn0_b"),
      p["sc1_node_w"], p["sc1_fn_w"], r2("sc1_fn_b"), r2("sc_bn1_g"),
      r2("sc_bn1_b"))


def _dec_body(zs_ref, zd_ref, w1, b1, w2, b2, w3, b3, w4, b4, out_ref):
    x = zs_ref[...][:, :8] * zd_ref[...][:, :8]
    x = _relu(jnp.dot(x.astype(jnp.bfloat16), w1[...],
                      preferred_element_type=jnp.float32) + b1[...])
    x = _relu(jnp.dot(x.astype(jnp.bfloat16), w2[...],
                      preferred_element_type=jnp.float32) + b2[...])
    x = _relu(jnp.dot(x.astype(jnp.bfloat16), w3[...],
                      preferred_element_type=jnp.float32) + b3[...])
    x = jnp.dot(x.astype(jnp.bfloat16), w4[...],
                preferred_element_type=jnp.float32) + b4[...]
    out_ref[...] = jax.nn.sigmoid(x)


def _decode(zsrc, zdst, p, blk):
    E = zsrc.shape[0]
    grid = E // blk
    full = lambda i: (0, 0)
    bf = lambda w: w.astype(jnp.bfloat16)
    return pl.pallas_call(
        _dec_body,
        grid=(grid,),
        in_specs=[
            pl.BlockSpec((blk, 16), lambda i: (i, 0)),
            pl.BlockSpec((blk, 16), lambda i: (i, 0)),
            pl.BlockSpec((8, 64), full),
            pl.BlockSpec((1, 64), full),
            pl.BlockSpec((64, 128), full),
            pl.BlockSpec((1, 128), full),
            pl.BlockSpec((128, 32), full),
            pl.BlockSpec((1, 32), full),
            pl.BlockSpec((32, 1), full),
            pl.BlockSpec((1, 1), full),
        ],
        out_specs=pl.BlockSpec((blk, 1), lambda i: (i, 0)),
        out_shape=jax.ShapeDtypeStruct((E, 1), jnp.float32),
    )(zsrc, zdst, bf(p["dec_w1"]), p["dec_b1"].reshape(1, -1),
      bf(p["dec_w2"]), p["dec_b2"].reshape(1, -1),
      bf(p["dec_w3"]), p["dec_b3"].reshape(1, -1),
      bf(p["dec_w4"]), p["dec_b4"].reshape(1, -1))


def kernel(fc_x, sc_x, fc_edge_attr, sc_edge_attr, edge_index, params):
    N = fc_x.shape[0]
    E = edge_index.shape[1]
    assert E % (_NW * _CH) == 0 and N % 16 == 0
    src_r = edge_index[0].reshape(E // _CH, _CH)
    dst_r = edge_index[1].reshape(E // _CH, _CH)
    dst = edge_index[1]

    g, h0 = _node0(fc_x, sc_x, params)
    gsrc, gdst = _sc_gather2(g, src_r, dst_r)
    attr_p = jnp.concatenate(
        [fc_edge_attr, sc_edge_attr], axis=1).reshape(E // 4, 16)
    wa, wc, bc, w11, s1 = _edge1_weights(params)
    scat_p = _edge1(gsrc.reshape(E // 4, 128), gdst.reshape(E // 4, 128),
                    attr_p, wa, wc, bc, w11, s1, blk=6400)
    scat = scat_p.reshape(E, 16)
    zacc = jnp.zeros((N, 16), jnp.float32)
    parts = _sc_scatter_add(scat, dst, zacc)
    zp = _node1(parts, h0, params)
    zsrc, zdst = _sc_gather2(zp, src_r, dst_r)
    return _decode(zsrc, zdst, params, blk=6400)


# speedup vs baseline: 7.9549x; 1.0274x over previous
import functools
import math

import jax
import jax.numpy as jnp
from jax import lax
from jax.experimental import pallas as pl
from jax.experimental.pallas import tpu as pltpu
from jax.experimental.pallas import tpu_sc as plsc

_NW = 32
_CH = 80
_BN = 1.0 / math.sqrt(1.0 + 1e-5)
_SC_PARAMS = pltpu.CompilerParams(use_tc_tiling_on_sc=False)


def _relu(x):
    return jnp.maximum(x, 0.0)


def _node0_body(fc_x, sc_x, fnw, snw, fewf, fews, g_ref, h0_ref):
    h0f = _relu(fc_x[...] @ fnw[...])
    h0s = _relu(sc_x[...] @ snw[...])
    gf = h0f @ fewf[...]
    gs = h0s @ fews[...]
    g_ref[...] = jnp.concatenate([gf, gs], axis=1)
    h0_ref[...] = jnp.concatenate([h0f, h0s], axis=1)


def _node0(fc_x, sc_x, p):
    N = fc_x.shape[0]
    return pl.pallas_call(
        _node0_body,
        out_shape=(
            jax.ShapeDtypeStruct((N, 32), jnp.float32),
            jax.ShapeDtypeStruct((N, 16), jnp.float32),
        ),
    )(fc_x, sc_x, p["fc0_node_w"], p["sc0_node_w"],
      p["fc0_fe_w"][:8], p["sc0_fe_w"][:8])


def _sc_gather2(tbl, src_r, dst_r):
    W = tbl.shape[1]
    n_ch_tot, ch = src_r.shape
    E = n_ch_tot * ch
    per_w = n_ch_tot // _NW
    D = 5
    mesh = plsc.VectorSubcoreMesh(core_axis_name="c", subcore_axis_name="s")
    out_t = jax.ShapeDtypeStruct((E, W), jnp.float32)

    @functools.partial(
        pl.kernel, mesh=mesh, compiler_params=_SC_PARAMS,
        out_type=(out_t, out_t),
        scratch_types=(
            [pltpu.VMEM((per_w, ch), jnp.int32)] * 2
            + [pltpu.VMEM((ch, W), jnp.float32)] * (2 * D)
            + [pltpu.SemaphoreType.DMA] * (3 * D)
        ),
    )
    def k(tbl_hbm, src_hbm, dst_hbm, oa_hbm, ob_hbm, *scr):
        si_all, di_all = scr[0], scr[1]
        bufs = scr[2:2 + 2 * D]
        gsems = scr[2 + 2 * D:2 + 4 * D]
        wsems = scr[2 + 4 * D:2 + 5 * D]
        wid = lax.axis_index("s") * 2 + lax.axis_index("c")
        c0 = wid * per_w
        pltpu.sync_copy(src_hbm.at[pl.ds(c0, per_w)], si_all)
        pltpu.sync_copy(dst_hbm.at[pl.ds(c0, per_w)], di_all)

        def gathers(ci, d):
            pltpu.make_async_copy(
                tbl_hbm.at[si_all.at[ci]], bufs[2 * d], gsems[2 * d]).start()
            pltpu.make_async_copy(
                tbl_hbm.at[di_all.at[ci]], bufs[2 * d + 1],
                gsems[2 * d + 1]).start()

        def wait_gathers(ci, d):
            pltpu.make_async_copy(
                tbl_hbm.at[si_all.at[ci]], bufs[2 * d], gsems[2 * d]).wait()
            pltpu.make_async_copy(
                tbl_hbm.at[di_all.at[ci]], bufs[2 * d + 1],
                gsems[2 * d + 1]).wait()

        def writes(ci, d):
            b = (c0 + ci) * ch
            pltpu.make_async_copy(
                bufs[2 * d], oa_hbm.at[pl.ds(b, ch)], wsems[d]).start()
            pltpu.make_async_copy(
                bufs[2 * d + 1], ob_hbm.at[pl.ds(b, ch)], wsems[d]).start()

        def wait_writes(ci, d):
            b = (c0 + ci) * ch
            pltpu.make_async_copy(
                bufs[2 * d], oa_hbm.at[pl.ds(b, ch)], wsems[d]).wait()
            pltpu.make_async_copy(
                bufs[2 * d + 1], ob_hbm.at[pl.ds(b, ch)], wsems[d]).wait()

        for d in range(D):
            gathers(d, d)

        @pl.loop(0, per_w, step=D)
        def _(j0):
            for d in range(D):
                j = j0 + d
                wait_gathers(j, d)
                writes(j, d)

                @pl.when(j + D < per_w)
                def _():
                    wait_writes(j, d)
                    gathers(j + D, d)

        for d in range(D):
            wait_writes(per_w - D + d, d)

    return k(tbl, src_r, dst_r)


def _edge1_body(gs_ref, gd_ref, attr_ref, wa, wc, bc, w11, s1, out_ref):
    t1 = _relu(attr_ref[...] @ wa[...])
    cc = t1 @ wc[...] + bc[...]
    e2 = _relu(gs_ref[...] + gd_ref[...] + cc)
    out_ref[...] = _relu(e2 @ w11[...]) + t1 @ s1[...]


def _edge1(gsrc, gdst, attr4, wa, wc, bc, w11, s1, blk):
    E = gsrc.shape[0]
    grid = E // blk
    full = lambda i: (0, 0)
    return pl.pallas_call(
        _edge1_body,
        grid=(grid,),
        in_specs=[
            pl.BlockSpec((blk, 32), lambda i: (i, 0)),
            pl.BlockSpec((blk, 32), lambda i: (i, 0)),
            pl.BlockSpec((blk, 4), lambda i: (i, 0)),
            pl.BlockSpec((4, 32), full),
            pl.BlockSpec((32, 32), full),
            pl.BlockSpec((1, 32), full),
            pl.BlockSpec((32, 16), full),
            pl.BlockSpec((32, 16), full),
        ],
        out_specs=pl.BlockSpec((blk, 16), lambda i: (i, 0)),
        out_shape=jax.ShapeDtypeStruct((E, 16), jnp.float32),
    )(gsrc, gdst, attr4, wa, wc, bc, w11, s1)


def _edge1_weights(p):
    wa = jnp.zeros((4, 32), jnp.float32)
    wa = wa.at[1:4, 0:8].set(p["sc0_edge_w"])
    wa = wa.at[0, 8].set(1.0)
    wa = wa.at[0, 9].set(-1.0)
    wa = wa.at[0, 10:18].set(p["fc0_edge_w"][0])
    wc = jnp.zeros((32, 32), jnp.float32)
    wc = wc.at[10:18, 0:16].set(p["fc0_fe_w"][8:])
    wc = wc.at[0:8, 16:32].set(p["sc0_fe_w"][8:])
    bc = jnp.concatenate([p["fc0_fe_b"], p["sc0_fe_b"]])
    w11 = jnp.zeros((32, 16), jnp.float32)
    w11 = w11.at[0:16, 10:12].set(p["fc1_edge_w"])
    w11 = w11.at[16:32, 12:14].set(p["sc1_edge_w"])
    s1 = jnp.zeros((32, 16), jnp.float32)
    s1 = s1.at[jnp.arange(10), jnp.arange(10)].set(1.0)
    return wa, wc, bc.reshape(1, 32), w11, s1


def _sc_scatter_add(scat, dst_r, zacc):
    E, W = scat.shape
    N = zacc.shape[0]
    ch = dst_r.shape[1]
    per_w = dst_r.shape[0] // _NW
    rows_w = N // 16
    D = 5
    mesh = plsc.VectorSubcoreMesh(core_axis_name="c", subcore_axis_name="s")

    @functools.partial(
        pl.kernel, mesh=mesh, compiler_params=_SC_PARAMS,
        out_type=jax.ShapeDtypeStruct((2, N, W), jnp.float32),
        scratch_types=(
            [pltpu.VMEM((per_w, ch), jnp.int32),
             pltpu.VMEM_SHARED((N, W), jnp.float32)]
            + [pltpu.VMEM((ch, W), jnp.float32)] * D
            + [pltpu.SemaphoreType.DMA] * (2 * D)
        ),
    )
    def k(scat_hbm, dst_hbm, z_hbm, out_hbm, *scr):
        di_all, acc = scr[0], scr[1]
        bufs = scr[2:2 + D]
        lsems = scr[2 + D:2 + 2 * D]
        ssems = scr[2 + 2 * D:2 + 3 * D]
        c = lax.axis_index("c")
        s = lax.axis_index("s")
        wid = s * 2 + c
        r0 = s * rows_w
        pltpu.sync_copy(z_hbm.at[pl.ds(r0, rows_w)], acc.at[pl.ds(r0, rows_w)])
        c0 = wid * per_w
        pltpu.sync_copy(dst_hbm.at[pl.ds(c0, per_w)], di_all)
        plsc.subcore_barrier()

        def load(ci, d):
            b = (c0 + ci) * ch
            pltpu.make_async_copy(
                scat_hbm.at[pl.ds(b, ch)], bufs[d], lsems[d]).start()

        def wait_load(ci, d):
            b = (c0 + ci) * ch
            pltpu.make_async_copy(
                scat_hbm.at[pl.ds(b, ch)], bufs[d], lsems[d]).wait()

        def sadd(ci, d):
            pltpu.async_copy(bufs[d], acc.at[di_all.at[ci]], ssems[d],
                             add=True)

        def wait_sadd(ci, d):
            pltpu.make_async_copy(
                bufs[d], acc.at[di_all.at[ci]], ssems[d]).wait()

        for d in range(D):
            load(d, d)

        @pl.loop(0, per_w, step=D)
        def _(j0):
            for d in range(D):
                j = j0 + d
                wait_load(j, d)
                sadd(j, d)

                @pl.when(j + D < per_w)
                def _():
                    wait_sadd(j, d)
                    load(j + D, d)

        for d in range(D):
            wait_sadd(per_w - D + d, d)

        plsc.subcore_barrier()
        pltpu.sync_copy(acc.at[pl.ds(r0, rows_w)],
                        out_hbm.at[c, pl.ds(r0, rows_w)])

    return k(scat, dst_r, zacc)


def _node1_body(parts_ref, h0_ref, few,
                ffn0w, ffn0b, fbn0g, fbn0b, fnw1, ffn1w, ffn1b, fbn1g, fbn1b,
                sfn0w, sfn0b, sbn0g, sbn0b, snw1, sfn1w, sfn1b, sbn1g, sbn1b,
                out_ref):
    parts = parts_ref[...]
    a = parts[0] + parts[1]
    h0 = h0_ref[...]
    wp = jnp.maximum(few[...], 0.0)
    wm = jnp.maximum(-few[...], 0.0)
    fc_aggr0 = a[:, 8:9] * wp + a[:, 9:10] * wm
    sc_aggr0 = a[:, 0:8]
    fc_a1 = a[:, 10:12]
    sc_a1 = a[:, 12:14]

    def half(aggr0, h0e, a1, fn0w, fn0b, bn0g, bn0b, nw1, fn1w, fn1b,
             bn1g, bn1b):
        n2 = _relu(jnp.concatenate([aggr0, h0e], axis=1) @ fn0w[...]
                   + fn0b[...])
        x1 = n2 * (bn0g[...] * _BN) + bn0b[...]
        h1 = _relu(x1 @ nw1[...])
        n2b = _relu(jnp.concatenate([a1, h1], axis=1) @ fn1w[...]
                    + fn1b[...])
        return n2b * (bn1g[...] * _BN) + bn1b[...]

    zfc = half(fc_aggr0, h0[:, :8], fc_a1, ffn0w, ffn0b, fbn0g, fbn0b,
               fnw1, ffn1w, ffn1b, fbn1g, fbn1b)
    zsc = half(sc_aggr0, h0[:, 8:], sc_a1, sfn0w, sfn0b, sbn0g, sbn0b,
               snw1, sfn1w, sfn1b, sbn1g, sbn1b)
    pad = jnp.zeros_like(zfc)
    out_ref[...] = jnp.concatenate([zfc, zsc, pad, pad], axis=1)


def _node1(parts, h0, p):
    N = h0.shape[0]
    r2 = lambda name: p[name].reshape(1, -1)
    return pl.pallas_call(
        _node1_body,
        out_shape=jax.ShapeDtypeStruct((N, 16), jnp.float32),
    )(parts, h0, p["fc0_edge_w"],
      p["fc0_fn_w"], r2("fc0_fn_b"), r2("fc_bn0_g"), r2("fc_bn0_b"),
      p["fc1_node_w"], p["fc1_fn_w"], r2("fc1_fn_b"), r2("fc_bn1_g"),
      r2("fc_bn1_b"),
      p["sc0_fn_w"], r2("sc0_fn_b"), r2("sc_bn0_g"), r2("sc_bn0_b"),
      p["sc1_node_w"], p["sc1_fn_w"], r2("sc1_fn_b"), r2("sc_bn1_g"),
      r2("sc_bn1_b"))


def _dec_body(zs_ref, zd_ref, w1, b1, w2, b2, w3, b3, w4, b4, out_ref):
    x = zs_ref[...][:, :8] * zd_ref[...][:, :8]
    x = _relu(jnp.dot(x.astype(jnp.bfloat16), w1[...],
                      preferred_element_type=jnp.float32) + b1[...])
    x = _relu(jnp.dot(x.astype(jnp.bfloat16), w2[...],
                      preferred_element_type=jnp.float32) + b2[...])
    x = _relu(jnp.dot(x.astype(jnp.bfloat16), w3[...],
                      preferred_element_type=jnp.float32) + b3[...])
    x = jnp.dot(x.astype(jnp.bfloat16), w4[...],
                preferred_element_type=jnp.float32) + b4[...]
    out_ref[...] = jax.nn.sigmoid(x)


def _decode(zsrc, zdst, p, blk):
    E = zsrc.shape[0]
    grid = E // blk
    full = lambda i: (0, 0)
    bf = lambda w: w.astype(jnp.bfloat16)
    return pl.pallas_call(
        _dec_body,
        grid=(grid,),
        in_specs=[
            pl.BlockSpec((blk, 16), lambda i: (i, 0)),
            pl.BlockSpec((blk, 16), lambda i: (i, 0)),
            pl.BlockSpec((8, 64), full),
            pl.BlockSpec((1, 64), full),
            pl.BlockSpec((64, 128), full),
            pl.BlockSpec((1, 128), full),
            pl.BlockSpec((128, 32), full),
            pl.BlockSpec((1, 32), full),
            pl.BlockSpec((32, 1), full),
            pl.BlockSpec((1, 1), full),
        ],
        out_specs=pl.BlockSpec((blk, 1), lambda i: (i, 0)),
        out_shape=jax.ShapeDtypeStruct((E, 1), jnp.float32),
    )(zsrc, zdst, bf(p["dec_w1"]), p["dec_b1"].reshape(1, -1),
      bf(p["dec_w2"]), p["dec_b2"].reshape(1, -1),
      bf(p["dec_w3"]), p["dec_b3"].reshape(1, -1),
      bf(p["dec_w4"]), p["dec_b4"].reshape(1, -1))


def kernel(fc_x, sc_x, fc_edge_attr, sc_edge_attr, edge_index, params):
    N = fc_x.shape[0]
    E = edge_index.shape[1]
    assert E % (_NW * _CH) == 0 and N % 16 == 0
    src_r = edge_index[0].reshape(E // _CH, _CH)
    dst_r = edge_index[1].reshape(E // _CH, _CH)
    dst = edge_index[1]

    g, h0 = _node0(fc_x, sc_x, params)
    gsrc, gdst = _sc_gather2(g, src_r, dst_r)
    attr4 = jnp.concatenate([fc_edge_attr, sc_edge_attr], axis=1)
    wa, wc, bc, w11, s1 = _edge1_weights(params)
    scat = _edge1(gsrc, gdst, attr4, wa, wc, bc, w11, s1, blk=6400)
    zacc = jnp.zeros((N, 16), jnp.float32)
    parts = _sc_scatter_add(scat, dst_r, zacc)
    zp = _node1(parts, h0, params)
    zsrc, zdst = _sc_gather2(zp, src_r, dst_r)
    return _decode(zsrc, zdst, params, blk=6400)


# speedup vs baseline: 11.1326x; 1.3995x over previous
import functools
import math

import jax
import jax.numpy as jnp
from jax import lax
from jax.experimental import pallas as pl
from jax.experimental.pallas import tpu as pltpu
from jax.experimental.pallas import tpu_sc as plsc

_NW = 32
_CH = 80
_BN = 1.0 / math.sqrt(1.0 + 1e-5)
_SC_PARAMS = pltpu.CompilerParams(use_tc_tiling_on_sc=False)


def _relu(x):
    return jnp.maximum(x, 0.0)


def _bd(m, k):
    out = jnp.zeros((m.shape[0] * k, m.shape[1] * k), m.dtype)
    for i in range(k):
        out = out.at[i * m.shape[0]:(i + 1) * m.shape[0],
                     i * m.shape[1]:(i + 1) * m.shape[1]].set(m)
    return out


def _node0_body(fc_x, sc_x, fnw, snw, fewf, fews, g_ref, h0_ref):
    h0f = _relu(fc_x[...] @ fnw[...])
    h0s = _relu(sc_x[...] @ snw[...])
    gf = h0f @ fewf[...]
    gs = h0s @ fews[...]
    g_ref[...] = jnp.concatenate([gf, gs], axis=1)
    h0_ref[...] = jnp.concatenate([h0f, h0s], axis=1)


def _node0(fc_x, sc_x, p):
    N = fc_x.shape[0]
    return pl.pallas_call(
        _node0_body,
        out_shape=(
            jax.ShapeDtypeStruct((N, 32), jnp.float32),
            jax.ShapeDtypeStruct((N, 16), jnp.float32),
        ),
    )(fc_x, sc_x, p["fc0_node_w"], p["sc0_node_w"],
      p["fc0_fe_w"][:8], p["sc0_fe_w"][:8])


def _sc_gather2(tbl, sp, dp):
    W = tbl.shape[1]
    E = sp.shape[0]
    per_w = E // _NW // _CH
    D = 5
    mesh = plsc.VectorSubcoreMesh(core_axis_name="c", subcore_axis_name="s")
    out_t = jax.ShapeDtypeStruct((E // 4, 128), jnp.float32)

    @functools.partial(
        pl.kernel, mesh=mesh, compiler_params=_SC_PARAMS,
        out_type=(out_t, out_t),
        scratch_types=(
            [pltpu.VMEM((per_w * _CH,), jnp.int32)] * 2
            + [pltpu.VMEM((_CH, W), jnp.float32)] * (2 * D)
            + [pltpu.SemaphoreType.DMA] * (3 * D)
        ),
    )
    def k(tbl_hbm, src_hbm, dst_hbm, oa_hbm, ob_hbm, *scr):
        si_all, di_all = scr[0], scr[1]
        bufs = scr[2:2 + 2 * D]
        gsems = scr[2 + 2 * D:2 + 4 * D]
        wsems = scr[2 + 4 * D:2 + 5 * D]
        wid = lax.axis_index("s") * 2 + lax.axis_index("c")
        e0 = wid * per_w * _CH
        pltpu.sync_copy(src_hbm.at[pl.ds(e0, per_w * _CH)], si_all)
        pltpu.sync_copy(dst_hbm.at[pl.ds(e0, per_w * _CH)], di_all)

        def idxs(ci):
            return (si_all.at[pl.ds(ci * _CH, _CH)],
                    di_all.at[pl.ds(ci * _CH, _CH)])

        def gathers(ci, d):
            si, di = idxs(ci)
            pltpu.make_async_copy(
                tbl_hbm.at[si], bufs[2 * d], gsems[2 * d]).start()
            pltpu.make_async_copy(
                tbl_hbm.at[di], bufs[2 * d + 1], gsems[2 * d + 1]).start()

        def wait_gathers(ci, d):
            si, di = idxs(ci)
            pltpu.make_async_copy(
                tbl_hbm.at[si], bufs[2 * d], gsems[2 * d]).wait()
            pltpu.make_async_copy(
                tbl_hbm.at[di], bufs[2 * d + 1], gsems[2 * d + 1]).wait()

        def writes(ci, d, wait):
            r0 = (e0 + ci * _CH) // 4
            for k_ in range(4):
                for b_, o_ in ((bufs[2 * d], oa_hbm),
                               (bufs[2 * d + 1], ob_hbm)):
                    cp = pltpu.make_async_copy(
                        b_.at[pl.ds(k_ * 20, 20)],
                        o_.at[pl.ds(r0, 20), pl.ds(32 * k_, 32)],
                        wsems[d])
                    cp.wait() if wait else cp.start()

        for d in range(D):
            gathers(d, d)

        @pl.loop(0, per_w, step=D)
        def _(j0):
            for d in range(D):
                j = j0 + d
                wait_gathers(j, d)
                writes(j, d, wait=False)

                @pl.when(j + D < per_w)
                def _():
                    writes(j, d, wait=True)
                    gathers(j + D, d)

        for d in range(D):
            writes(per_w - D + d, d, wait=True)

    return k(tbl, sp, dp)


def _edge1_body(gs_ref, gd_ref, attr_ref, wa, wc, bc, w11, s1, out_ref):
    t1 = _relu(attr_ref[...] @ wa[...])
    cc = t1 @ wc[...] + bc[...]
    e2 = _relu(gs_ref[...] + gd_ref[...] + cc)
    out_ref[...] = _relu(e2 @ w11[...]) + t1 @ s1[...]


def _edge1(gsrc, gdst, attr_p, wa, wc, bc, w11, s1, blk):
    E4 = gsrc.shape[0]
    b4 = blk // 4
    grid = E4 // b4
    full = lambda i: (0, 0)
    return pl.pallas_call(
        _edge1_body,
        grid=(grid,),
        in_specs=[
            pl.BlockSpec((b4, 128), lambda i: (i, 0)),
            pl.BlockSpec((b4, 128), lambda i: (i, 0)),
            pl.BlockSpec((b4, 16), lambda i: (i, 0)),
            pl.BlockSpec((16, 128), full),
            pl.BlockSpec((128, 128), full),
            pl.BlockSpec((1, 128), full),
            pl.BlockSpec((128, 128), full),
            pl.BlockSpec((128, 128), full),
        ],
        out_specs=pl.BlockSpec((b4, 128), lambda i: (i, 0)),
        out_shape=jax.ShapeDtypeStruct((E4, 128), jnp.float32),
    )(gsrc, gdst, attr_p, wa, wc, bc, w11, s1)


def _edge1_weights(p):
    wa = jnp.zeros((4, 32), jnp.float32)
    wa = wa.at[1:4, 0:8].set(p["sc0_edge_w"])
    wa = wa.at[0, 8].set(1.0)
    wa = wa.at[0, 9].set(-1.0)
    wa = wa.at[0, 10:18].set(p["fc0_edge_w"][0])
    wc = jnp.zeros((32, 32), jnp.float32)
    wc = wc.at[10:18, 0:16].set(p["fc0_fe_w"][8:])
    wc = wc.at[0:8, 16:32].set(p["sc0_fe_w"][8:])
    bc = jnp.concatenate([p["fc0_fe_b"], p["sc0_fe_b"]])
    w11 = jnp.zeros((32, 32), jnp.float32)
    w11 = w11.at[0:16, 10:12].set(p["fc1_edge_w"])
    w11 = w11.at[16:32, 12:14].set(p["sc1_edge_w"])
    s1 = jnp.zeros((32, 32), jnp.float32)
    s1 = s1.at[jnp.arange(10), jnp.arange(10)].set(1.0)
    return (_bd(wa, 4), _bd(wc, 4), jnp.tile(bc, 4).reshape(1, 128),
            _bd(w11, 4), _bd(s1, 4))


def _sc_scatter_add(scat_p, dp, zacc):
    W = 32
    E = dp.shape[0]
    N = zacc.shape[0]
    per_w = E // _NW // _CH
    rows_w = N // 16
    D = 5
    mesh = plsc.VectorSubcoreMesh(core_axis_name="c", subcore_axis_name="s")

    @functools.partial(
        pl.kernel, mesh=mesh, compiler_params=_SC_PARAMS,
        out_type=jax.ShapeDtypeStruct((2, N, W), jnp.float32),
        scratch_types=(
            [pltpu.VMEM((per_w * _CH,), jnp.int32),
             pltpu.VMEM_SHARED((N, W), jnp.float32)]
            + [pltpu.VMEM((_CH, W), jnp.float32)] * D
            + [pltpu.SemaphoreType.DMA] * (2 * D)
        ),
    )
    def k(scat_hbm, dst_hbm, z_hbm, out_hbm, *scr):
        di_all, acc = scr[0], scr[1]
        bufs = scr[2:2 + D]
        lsems = scr[2 + D:2 + 2 * D]
        ssems = scr[2 + 2 * D:2 + 3 * D]
        c = lax.axis_index("c")
        s = lax.axis_index("s")
        wid = s * 2 + c
        r0 = s * rows_w
        pltpu.sync_copy(z_hbm.at[pl.ds(r0, rows_w)], acc.at[pl.ds(r0, rows_w)])
        e0 = wid * per_w * _CH
        pltpu.sync_copy(dst_hbm.at[pl.ds(e0, per_w * _CH)], di_all)
        plsc.subcore_barrier()

        def load(ci, d, wait):
            p0 = (e0 + ci * _CH) // 4
            for k_ in range(4):
                cp = pltpu.make_async_copy(
                    scat_hbm.at[pl.ds(p0, 20), pl.ds(32 * k_, 32)],
                    bufs[d].at[pl.ds(k_ * 20, 20)],
                    lsems[d])
                cp.wait() if wait else cp.start()

        def sadd(ci, d):
            pltpu.async_copy(bufs[d],
                             acc.at[di_all.at[pl.ds(ci * _CH, _CH)]],
                             ssems[d], add=True)

        def wait_sadd(ci, d):
            pltpu.make_async_copy(
                bufs[d], acc.at[di_all.at[pl.ds(ci * _CH, _CH)]],
                ssems[d]).wait()

        for d in range(D):
            load(d, d, wait=False)

        @pl.loop(0, per_w, step=D)
        def _(j0):
            for d in range(D):
                j = j0 + d
                load(j, d, wait=True)
                sadd(j, d)

                @pl.when(j + D < per_w)
                def _():
                    wait_sadd(j, d)
                    load(j + D, d, wait=False)

        for d in range(D):
            wait_sadd(per_w - D + d, d)

        plsc.subcore_barrier()
        pltpu.sync_copy(acc.at[pl.ds(r0, rows_w)],
                        out_hbm.at[c, pl.ds(r0, rows_w)])

    return k(scat_p, dp, zacc)


def _node1_body(parts_ref, h0_ref, few,
                ffn0w, ffn0b, fbn0g, fbn0b, fnw1, ffn1w, ffn1b, fbn1g, fbn1b,
                sfn0w, sfn0b, sbn0g, sbn0b, snw1, sfn1w, sfn1b, sbn1g, sbn1b,
                out_ref):
    parts = parts_ref[...]
    a = parts[0] + parts[1]
    h0 = h0_ref[...]
    wp = jnp.maximum(few[...], 0.0)
    wm = jnp.maximum(-few[...], 0.0)
    fc_aggr0 = a[:, 8:9] * wp + a[:, 9:10] * wm
    sc_aggr0 = a[:, 0:8]
    fc_a1 = a[:, 10:12]
    sc_a1 = a[:, 12:14]

    def half(aggr0, h0e, a1, fn0w, fn0b, bn0g, bn0b, nw1, fn1w, fn1b,
             bn1g, bn1b):
        n2 = _relu(jnp.concatenate([aggr0, h0e], axis=1) @ fn0w[...]
                   + fn0b[...])
        x1 = n2 * (bn0g[...] * _BN) + bn0b[...]
        h1 = _relu(x1 @ nw1[...])
        n2b = _relu(jnp.concatenate([a1, h1], axis=1) @ fn1w[...]
                    + fn1b[...])
        return n2b * (bn1g[...] * _BN) + bn1b[...]

    zfc = half(fc_aggr0, h0[:, :8], fc_a1, ffn0w, ffn0b, fbn0g, fbn0b,
               fnw1, ffn1w, ffn1b, fbn1g, fbn1b)
    zsc = half(sc_aggr0, h0[:, 8:], sc_a1, sfn0w, sfn0b, sbn0g, sbn0b,
               snw1, sfn1w, sfn1b, sbn1g, sbn1b)
    pad = jnp.zeros_like(zfc)
    out_ref[...] = jnp.concatenate([zfc, zsc] + [pad] * 6, axis=1)


def _node1(parts, h0, p):
    N = h0.shape[0]
    r2 = lambda name: p[name].reshape(1, -1)
    return pl.pallas_call(
        _node1_body,
        out_shape=jax.ShapeDtypeStruct((N, 32), jnp.float32),
    )(parts, h0, p["fc0_edge_w"],
      p["fc0_fn_w"], r2("fc0_fn_b"), r2("fc_bn0_g"), r2("fc_bn0_b"),
      p["fc1_node_w"], p["fc1_fn_w"], r2("fc1_fn_b"), r2("fc_bn1_g"),
      r2("fc_bn1_b"),
      p["sc0_fn_w"], r2("sc0_fn_b"), r2("sc_bn0_g"), r2("sc_bn0_b"),
      p["sc1_node_w"], p["sc1_fn_w"], r2("sc1_fn_b"), r2("sc_bn1_g"),
      r2("sc_bn1_b"))


def _dec_body(zs_ref, zd_ref, w1, b1, w2, b2, w3, b3, w4, b4, out_ref):
    x = zs_ref[...] * zd_ref[...]
    x = _relu(jnp.dot(x.astype(jnp.bfloat16), w1[...],
                      preferred_element_type=jnp.float32) + b1[...])
    x = _relu(jnp.dot(x.astype(jnp.bfloat16), w2[...],
                      preferred_element_type=jnp.float32) + b2[...])
    x = _relu(jnp.dot(x.astype(jnp.bfloat16), w3[...],
                      preferred_element_type=jnp.float32) + b3[...])
    x = jnp.dot(x.astype(jnp.bfloat16), w4[...],
                preferred_element_type=jnp.float32) + b4[...]
    out_ref[...] = jax.nn.sigmoid(x)


def _decode(zsrc_p, zdst_p, p, blk):
    E4 = zsrc_p.shape[0]
    b4 = blk // 4
    grid = E4 // b4
    full = lambda i: (0, 0)
    bf = lambda w: w.astype(jnp.bfloat16)
    w1p = jnp.zeros((32, 64), jnp.float32).at[:8].set(p["dec_w1"])
    return pl.pallas_call(
        _dec_body,
        grid=(grid,),
        in_specs=[
            pl.BlockSpec((b4, 128), lambda i: (i, 0)),
            pl.BlockSpec((b4, 128), lambda i: (i, 0)),
            pl.BlockSpec((128, 256), full),
            pl.BlockSpec((1, 256), full),
            pl.BlockSpec((256, 512), full),
            pl.BlockSpec((1, 512), full),
            pl.BlockSpec((512, 128), full),
            pl.BlockSpec((1, 128), full),
            pl.BlockSpec((128, 4), full),
            pl.BlockSpec((1, 4), full),
        ],
        out_specs=pl.BlockSpec((b4, 4), lambda i: (i, 0)),
        out_shape=jax.ShapeDtypeStruct((E4, 4), jnp.float32),
    )(zsrc_p, zdst_p,
      bf(_bd(w1p, 4)), jnp.tile(p["dec_b1"], 4).reshape(1, 256),
      bf(_bd(p["dec_w2"], 4)), jnp.tile(p["dec_b2"], 4).reshape(1, 512),
      bf(_bd(p["dec_w3"], 4)), jnp.tile(p["dec_b3"], 4).reshape(1, 128),
      bf(_bd(p["dec_w4"], 4)), jnp.tile(p["dec_b4"], 4).reshape(1, 4))


def kernel(fc_x, sc_x, fc_edge_attr, sc_edge_attr, edge_index, params):
    N = fc_x.shape[0]
    E = edge_index.shape[1]
    assert E % (_NW * _CH) == 0 and N % 16 == 0
    perm = lambda v: v.reshape(E // _CH, 20, 4).transpose(0, 2, 1).reshape(E)
    sp = perm(edge_index[0])
    dp = perm(edge_index[1])

    g, h0 = _node0(fc_x, sc_x, params)
    gsrc_p, gdst_p = _sc_gather2(g, sp, dp)
    attr_p = jnp.concatenate(
        [fc_edge_attr, sc_edge_attr], axis=1).reshape(E // 4, 16)
    wa, wc, bc, w11, s1 = _edge1_weights(params)
    scat_p = _edge1(gsrc_p, gdst_p, attr_p, wa, wc, bc, w11, s1, blk=6400)
    zacc = jnp.zeros((N, 32), jnp.float32)
    parts = _sc_scatter_add(scat_p, dp, zacc)
    zp = _node1(parts, h0, params)
    zsrc_p, zdst_p = _sc_gather2(zp, sp, dp)
    out_p = _decode(zsrc_p, zdst_p, params, blk=6400)
    return out_p.reshape(E, 1)
